# Initial kernel scaffold; baseline (speedup 1.0000x reference)
import jax
import jax.numpy as jnp
from jax.experimental import pallas as pl


def kernel(x, table):
    raise NotImplementedError("write your pallas kernel here")


# speedup vs baseline: 3.9807x; 3.9807x over previous
import functools

import jax
import jax.numpy as jnp
from jax import lax
from jax.experimental import pallas as pl
from jax.experimental.pallas import tpu as pltpu, tpu_sc as plsc

PHONEME_SIZE = 1000
D = 256
N = 4096 * 200
NW = 32
PER_W = N // NW
CHUNK = 128
N_CHUNKS = PER_W // CHUNK


def _make_kernel():
    mesh = plsc.VectorSubcoreMesh(core_axis_name="c", subcore_axis_name="s")

    @functools.partial(
        pl.kernel,
        mesh=mesh,
        out_type=jax.ShapeDtypeStruct((N, D), jnp.float32),
        scratch_types=[
            pltpu.VMEM((CHUNK,), jnp.int32),
            pltpu.VMEM((CHUNK, D), jnp.float32),
            pltpu.SemaphoreType.DMA,
        ],
    )
    def k(x_hbm, table_hbm, out_hbm, idx_v, rows_v, sem):
        wid = lax.axis_index("s") * 2 + lax.axis_index("c")
        base = wid * PER_W

        def body(i, carry):
            off = base + i * CHUNK
            pltpu.sync_copy(x_hbm.at[pl.ds(off, CHUNK)], idx_v)
            pltpu.async_copy(table_hbm.at[idx_v], rows_v, sem).wait()
            pltpu.sync_copy(rows_v, out_hbm.at[pl.ds(off, CHUNK)])
            return carry

        lax.fori_loop(0, N_CHUNKS, body, 0)

    return k


_kernel = _make_kernel()


def kernel(x, table):
    flat = x.reshape(N).astype(jnp.int32)
    out = _kernel(flat, table)
    return out.reshape(x.shape[0], x.shape[1], D)


# speedup vs baseline: 4.7871x; 1.2026x over previous
import functools

import jax
import jax.numpy as jnp
from jax import lax
from jax.experimental import pallas as pl
from jax.experimental.pallas import tpu as pltpu, tpu_sc as plsc

D = 256
N = 4096 * 200
NW = 32
PER_W = N // NW
CHUNK = 128
N_CHUNKS = PER_W // CHUNK


def _make_kernel():
    mesh = plsc.VectorSubcoreMesh(core_axis_name="c", subcore_axis_name="s")

    @functools.partial(
        pl.kernel,
        mesh=mesh,
        out_type=jax.ShapeDtypeStruct((N, D), jnp.float32),
        scratch_types=[
            pltpu.VMEM((N_CHUNKS, CHUNK), jnp.int32),
            pltpu.VMEM((2, CHUNK, D), jnp.float32),
            pltpu.SemaphoreType.DMA,
            pltpu.SemaphoreType.DMA,
        ],
    )
    def k(x_hbm, table_hbm, out_hbm, idx_v, rows, gsem, ssem):
        wid = lax.axis_index("s") * 2 + lax.axis_index("c")
        base = wid * PER_W
        pltpu.sync_copy(x_hbm.at[wid], idx_v)

        def gather(i, b):
            return pltpu.make_async_copy(
                table_hbm.at[idx_v.at[i]], rows.at[b], gsem)

        def scatter(i, b):
            return pltpu.make_async_copy(
                rows.at[b], out_hbm.at[pl.ds(base + i * CHUNK, CHUNK)], ssem)

        gather(0, 0).start()

        @pl.loop(0, N_CHUNKS, step=2)
        def _(i):
            for b in range(2):
                ci = i + b
                nb = 1 - b

                @pl.when(ci + 1 < N_CHUNKS)
                def _():
                    @pl.when(ci >= 1)
                    def _():
                        scatter(ci - 1, nb).wait()
                    gather(ci + 1, nb).start()

                gather(ci, b).wait()
                scatter(ci, b).start()

        scatter(N_CHUNKS - 2, 0).wait()
        scatter(N_CHUNKS - 1, 1).wait()

    return k


_kernel = _make_kernel()


def kernel(x, table):
    flat = x.reshape(NW, N_CHUNKS, CHUNK).astype(jnp.int32)
    out = _kernel(flat, table)
    return out.reshape(x.shape[0], x.shape[1], D)


# speedup vs baseline: 4.8091x; 1.0046x over previous
import functools

import jax
import jax.numpy as jnp
from jax import lax
from jax.experimental import pallas as pl
from jax.experimental.pallas import tpu as pltpu, tpu_sc as plsc

D = 256
N = 4096 * 200
NW = 32
PER_W = N // NW
CHUNK = 128
N_CHUNKS = PER_W // CHUNK
NBUF = 3


def _make_kernel():
    mesh = plsc.VectorSubcoreMesh(core_axis_name="c", subcore_axis_name="s")

    @functools.partial(
        pl.kernel,
        mesh=mesh,
        out_type=jax.ShapeDtypeStruct((N, D), jnp.float32),
        scratch_types=[
            pltpu.VMEM((N_CHUNKS, CHUNK), jnp.int32),
            pltpu.VMEM((NBUF, CHUNK, D), jnp.float32),
            pltpu.SemaphoreType.DMA,
            pltpu.SemaphoreType.DMA,
        ],
    )
    def k(x_hbm, table_hbm, out_hbm, idx_v, rows, gsem, ssem):
        wid = lax.axis_index("s") * 2 + lax.axis_index("c")
        base = wid * PER_W
        pltpu.sync_copy(x_hbm.at[wid], idx_v)

        def gather(i, b):
            return pltpu.make_async_copy(
                table_hbm.at[idx_v.at[i]], rows.at[b], gsem)

        def scatter(i, b):
            return pltpu.make_async_copy(
                rows.at[b], out_hbm.at[pl.ds(base + i * CHUNK, CHUNK)], ssem)

        for j in range(NBUF - 1):
            gather(j, j).start()

        MAIN = (N_CHUNKS // NBUF) * NBUF

        @pl.loop(0, MAIN, step=NBUF)
        def _(i):
            for b in range(NBUF):
                ci = i + b
                nb = (ci + NBUF - 1) % NBUF

                @pl.when(ci + NBUF - 1 < N_CHUNKS)
                def _():
                    @pl.when(ci >= 1)
                    def _():
                        scatter(ci - 1, nb).wait()
                    gather(ci + NBUF - 1, nb).start()

                gather(ci, b).wait()
                scatter(ci, b).start()

        for ci in range(MAIN, N_CHUNKS):
            gather(ci, ci % NBUF).wait()
            scatter(ci, ci % NBUF).start()

        for ci in range(N_CHUNKS - NBUF, N_CHUNKS):
            scatter(ci, ci % NBUF).wait()

    return k


_kernel = _make_kernel()


def kernel(x, table):
    flat = x.reshape(NW, N_CHUNKS, CHUNK).astype(jnp.int32)
    out = _kernel(flat, table)
    return out.reshape(x.shape[0], x.shape[1], D)


# speedup vs baseline: 5.5882x; 1.1620x over previous
import functools

import jax
import jax.numpy as jnp
from jax import lax
from jax.experimental import pallas as pl
from jax.experimental.pallas import tpu as pltpu, tpu_sc as plsc

D = 256
N = 4096 * 200
NW = 32
PER_W = N // NW
CHUNK = 128
N_CHUNKS = PER_W // CHUNK
NBUF = 3
NCOPY = 8
V = 1000


def _make_kernel():
    mesh = plsc.VectorSubcoreMesh(core_axis_name="c", subcore_axis_name="s")

    @functools.partial(
        pl.kernel,
        mesh=mesh,
        out_type=jax.ShapeDtypeStruct((N, D), jnp.float32),
        scratch_types=[
            pltpu.VMEM((N_CHUNKS, CHUNK), jnp.int32),
            pltpu.VMEM((NBUF, CHUNK, D), jnp.float32),
            pltpu.SemaphoreType.DMA,
            pltpu.SemaphoreType.DMA,
        ],
    )
    def k(x_hbm, table_hbm, out_hbm, idx_v, rows, gsem, ssem):
        wid = lax.axis_index("s") * 2 + lax.axis_index("c")
        base = wid * PER_W
        pltpu.sync_copy(x_hbm.at[wid], idx_v)

        off = (lax.rem(wid, NCOPY) * V).astype(jnp.int32)

        @pl.loop(0, N_CHUNKS)
        def _(ci):
            for j in range(CHUNK // 16):
                sl = pl.ds(j * 16, 16)
                idx_v[ci, sl] = idx_v[ci, sl] + off

        def gather(i, b):
            return pltpu.make_async_copy(
                table_hbm.at[idx_v.at[i]], rows.at[b], gsem)

        def scatter(i, b):
            return pltpu.make_async_copy(
                rows.at[b], out_hbm.at[pl.ds(base + i * CHUNK, CHUNK)], ssem)

        for j in range(NBUF - 1):
            gather(j, j).start()

        MAIN = (N_CHUNKS // NBUF) * NBUF

        @pl.loop(0, MAIN, step=NBUF)
        def _(i):
            for b in range(NBUF):
                ci = i + b
                nb = (ci + NBUF - 1) % NBUF

                @pl.when(ci + NBUF - 1 < N_CHUNKS)
                def _():
                    @pl.when(ci >= 1)
                    def _():
                        scatter(ci - 1, nb).wait()
                    gather(ci + NBUF - 1, nb).start()

                gather(ci, b).wait()
                scatter(ci, b).start()

        for ci in range(MAIN, N_CHUNKS):
            gather(ci, ci % NBUF).wait()
            scatter(ci, ci % NBUF).start()

        for ci in range(N_CHUNKS - NBUF, N_CHUNKS):
            scatter(ci, ci % NBUF).wait()

    return k


_kernel = _make_kernel()


def kernel(x, table):
    flat = x.reshape(NW, N_CHUNKS, CHUNK).astype(jnp.int32)
    table_rep = jnp.concatenate([table] * NCOPY, axis=0)
    out = _kernel(flat, table_rep)
    return out.reshape(x.shape[0], x.shape[1], D)
